# baseline (speedup 1.0000x reference)
import jax
import jax.numpy as jnp
from jax.experimental import pallas as pl
from jax.experimental.pallas import tpu as pltpu

N = 2708
BM = 512
G = 6
NPAD = BM * G

_LAYERS = (
    (1433, 1024, True),
    (1024, 512, True),
    (512, 256, True),
    (256, 16, False),
    (16, 256, True),
    (256, 512, True),
    (512, 1024, True),
    (1024, 1433, False),
)


def _body(x_ref, adj_ref, *rest):
    w_refs = rest[0:8]
    b_refs = rest[8:16]
    z_ref, xr_ref = rest[16:18]
    ha_ref, hb_ref, s_ref, zs_ref = rest[18:22]

    lp = pl.program_id(0)
    i = pl.program_id(1)
    row = i * BM

    srcs = (None, ha_ref, hb_ref, ha_ref, zs_ref, hb_ref, ha_ref, hb_ref)
    dsts = (ha_ref, hb_ref, ha_ref, None, hb_ref, ha_ref, hb_ref, None)

    for l, (k, n, relu) in enumerate(_LAYERS):
        @pl.when(lp == 2 * l)
        def _(l=l, k=k, n=n):
            if l == 0:
                h_blk = x_ref[...]
            else:
                h_blk = srcs[l][pl.ds(row, BM), :k]
            s32 = jnp.dot(h_blk, w_refs[l][...],
                          preferred_element_type=jnp.float32)
            s_ref[pl.ds(row, BM), :n] = s32.astype(jnp.bfloat16)

        @pl.when(lp == 2 * l + 1)
        def _(l=l, n=n, relu=relu):
            acc = jnp.dot(adj_ref[...], s_ref[:N, :n],
                          preferred_element_type=jnp.float32)
            acc = acc + b_refs[l][...]
            if relu:
                acc = jnp.maximum(acc, 0.0)
            if l == 3:
                z_ref[...] = acc
                zs_ref[pl.ds(row, BM), :] = acc.astype(jnp.bfloat16)
            elif l == 7:
                xr_ref[...] = acc
            else:
                dsts[l][pl.ds(row, BM), :n] = acc.astype(jnp.bfloat16)


def kernel(x, adj, We1, be1, We2, be2, We3, be3, Wez, bez,
           Wd1, bd1, Wd2, bd2, Wd3, bd3, Wdf, bdf):
    bf = jnp.bfloat16
    ws = [w.astype(bf) for w in (We1, We2, We3, Wez, Wd1, Wd2, Wd3, Wdf)]
    bs = [b.reshape(1, -1) for b in (be1, be2, be3, bez, bd1, bd2, bd3, bdf)]

    in_specs = [
        pl.BlockSpec((BM, 1433), lambda lp, i: (jnp.where(lp == 0, i, 0), 0)),
        pl.BlockSpec((BM, N), lambda lp, i: (jnp.where(lp % 2 == 1, i, 0), 0)),
    ]
    in_specs += [pl.BlockSpec(w.shape, lambda lp, i: (0, 0)) for w in ws]
    in_specs += [pl.BlockSpec(b.shape, lambda lp, i: (0, 0)) for b in bs]

    out_specs = [
        pl.BlockSpec((BM, 16), lambda lp, i: (
            jnp.where(lp < 7, 0, jnp.where(lp == 7, i, G - 1)), 0)),
        pl.BlockSpec((BM, 1433), lambda lp, i: (jnp.where(lp == 15, i, 0), 0)),
    ]

    z, xr = pl.pallas_call(
        _body,
        grid=(16, G),
        in_specs=in_specs,
        out_specs=out_specs,
        out_shape=[
            jax.ShapeDtypeStruct((N, 16), jnp.float32),
            jax.ShapeDtypeStruct((N, 1433), jnp.float32),
        ],
        scratch_shapes=[
            pltpu.VMEM((NPAD, 1024), bf),
            pltpu.VMEM((NPAD, 1024), bf),
            pltpu.VMEM((NPAD, 1433), bf),
            pltpu.VMEM((NPAD, 16), bf),
        ],
    )(x.astype(bf), adj.astype(bf), *ws, *bs)
    return (z, xr)


# speedup vs baseline: 1.0674x; 1.0674x over previous
import jax
import jax.numpy as jnp
from jax.experimental import pallas as pl
from jax.experimental.pallas import tpu as pltpu

N = 2708
BM = 352
G = 8
NPAD = BM * G

_LAYERS = (
    (1433, 1024, True),
    (1024, 512, True),
    (512, 256, True),
    (256, 16, False),
    (16, 256, True),
    (256, 512, True),
    (512, 1024, True),
    (1024, 1433, False),
)


def _body(x_ref, adj_ref, *rest):
    w_refs = rest[0:8]
    b_refs = rest[8:16]
    z_ref, xr_ref = rest[16:18]
    adj_scr, ha_ref, hb_ref, s_ref, zs_ref = rest[18:23]

    lp = pl.program_id(0)
    i = pl.program_id(1)
    row = i * BM

    srcs = (None, ha_ref, hb_ref, ha_ref, zs_ref, hb_ref, ha_ref, hb_ref)
    dsts = (ha_ref, hb_ref, ha_ref, None, hb_ref, ha_ref, hb_ref, None)

    for l, (k, n, relu) in enumerate(_LAYERS):
        @pl.when(lp == 2 * l)
        def _(l=l, k=k, n=n):
            if l == 0:
                adj_scr[pl.ds(row, BM), :] = adj_ref[...].astype(jnp.bfloat16)
                h_blk = x_ref[...]
            else:
                h_blk = srcs[l][pl.ds(row, BM), :k]
            s32 = jnp.dot(h_blk, w_refs[l][...],
                          preferred_element_type=jnp.float32)
            s_ref[pl.ds(row, BM), :n] = s32.astype(jnp.bfloat16)

        @pl.when(lp == 2 * l + 1)
        def _(l=l, n=n, relu=relu):
            acc = jnp.dot(adj_scr[pl.ds(row, BM), :], s_ref[:N, :n],
                          preferred_element_type=jnp.float32)
            acc = acc + b_refs[l][...]
            if relu:
                acc = jnp.maximum(acc, 0.0)
            if l == 3:
                z_ref[...] = acc
                zs_ref[pl.ds(row, BM), :] = acc.astype(jnp.bfloat16)
            elif l == 7:
                xr_ref[...] = acc
            else:
                dsts[l][pl.ds(row, BM), :n] = acc.astype(jnp.bfloat16)


def kernel(x, adj, We1, be1, We2, be2, We3, be3, Wez, bez,
           Wd1, bd1, Wd2, bd2, Wd3, bd3, Wdf, bdf):
    bf = jnp.bfloat16
    ws = [w.astype(bf) for w in (We1, We2, We3, Wez, Wd1, Wd2, Wd3, Wdf)]
    bs = [b.reshape(1, -1) for b in (be1, be2, be3, bez, bd1, bd2, bd3, bdf)]

    in_specs = [
        pl.BlockSpec((BM, 1433),
                     lambda lp, i: (jnp.where(lp == 0, i, G - 1), 0)),
        pl.BlockSpec((BM, N),
                     lambda lp, i: (jnp.where(lp == 0, i, G - 1), 0)),
    ]
    in_specs += [pl.BlockSpec(w.shape, lambda lp, i: (0, 0)) for w in ws]
    in_specs += [pl.BlockSpec(b.shape, lambda lp, i: (0, 0)) for b in bs]

    out_specs = [
        pl.BlockSpec((BM, 16), lambda lp, i: (
            jnp.where(lp < 7, 0, jnp.where(lp == 7, i, G - 1)), 0)),
        pl.BlockSpec((BM, 1433), lambda lp, i: (jnp.where(lp == 15, i, 0), 0)),
    ]

    z, xr = pl.pallas_call(
        _body,
        grid=(16, G),
        in_specs=in_specs,
        out_specs=out_specs,
        out_shape=[
            jax.ShapeDtypeStruct((N, 16), jnp.float32),
            jax.ShapeDtypeStruct((N, 1433), jnp.float32),
        ],
        scratch_shapes=[
            pltpu.VMEM((NPAD, N), bf),
            pltpu.VMEM((NPAD, 1024), bf),
            pltpu.VMEM((NPAD, 1024), bf),
            pltpu.VMEM((NPAD, 1433), bf),
            pltpu.VMEM((NPAD, 16), bf),
        ],
    )(x.astype(bf), adj, *ws, *bs)
    return (z, xr)
